# baseline (speedup 1.0000x reference)
import functools

import jax
import jax.numpy as jnp
from jax import lax
from jax.experimental import pallas as pl
from jax.experimental.pallas import tpu as pltpu
from jax.experimental.pallas import tpu_sc as plsc

NC = 2
NS = 16
NW = NC * NS

CHUNK = 80


def kernel(idx, table):
    B, H = idx.shape
    V, D = table.shape
    n = B * H
    per_w = n // NW
    n_chunks = per_w // CHUNK

    mesh = plsc.VectorSubcoreMesh(
        core_axis_name="c", subcore_axis_name="s",
        num_cores=NC, num_subcores=NS,
    )

    @functools.partial(
        pl.kernel,
        out_type=jax.ShapeDtypeStruct((n, D), jnp.float32),
        mesh=mesh,
        scratch_types=[
            pltpu.VMEM((per_w,), jnp.int32),
            pltpu.VMEM((CHUNK, D), jnp.float32),
            pltpu.SemaphoreType.DMA,
        ],
        compiler_params=pltpu.CompilerParams(use_tc_tiling_on_sc=False),
    )
    def gather_k(idx_hbm, table_hbm, out_hbm, idx_v, rows_v, sem):
        wid = lax.axis_index("s") * NC + lax.axis_index("c")
        base = wid * per_w
        pltpu.sync_copy(idx_hbm.at[pl.ds(base, per_w)], idx_v)

        @pl.loop(0, n_chunks)
        def _(i):
            pltpu.async_copy(
                table_hbm.at[idx_v.at[pl.ds(i * CHUNK, CHUNK)]], rows_v, sem
            ).wait()
            pltpu.sync_copy(rows_v, out_hbm.at[pl.ds(base + i * CHUNK, CHUNK)])

    out = gather_k(idx.reshape(n).astype(jnp.int32), table)
    return out.reshape(B, H, D)


# speedup vs baseline: 1.0115x; 1.0115x over previous
import functools

import jax
import jax.numpy as jnp
from jax import lax
from jax.experimental import pallas as pl
from jax.experimental.pallas import tpu as pltpu
from jax.experimental.pallas import tpu_sc as plsc

NC = 2
NS = 16
NW = NC * NS

CHUNK = 40


def kernel(idx, table):
    B, H = idx.shape
    V, D = table.shape
    n = B * H
    per_w = n // NW
    n_chunks = per_w // CHUNK

    mesh = plsc.VectorSubcoreMesh(
        core_axis_name="c", subcore_axis_name="s",
        num_cores=NC, num_subcores=NS,
    )

    @functools.partial(
        pl.kernel,
        out_type=jax.ShapeDtypeStruct((n, D), jnp.float32),
        mesh=mesh,
        scratch_types=[
            pltpu.VMEM((per_w,), jnp.int32),
            pltpu.VMEM((CHUNK, D), jnp.float32),
            pltpu.VMEM((CHUNK, D), jnp.float32),
            pltpu.SemaphoreType.DMA,
            pltpu.SemaphoreType.DMA,
        ],
        compiler_params=pltpu.CompilerParams(use_tc_tiling_on_sc=False),
    )
    def gather_k(idx_hbm, table_hbm, out_hbm, idx_v, buf0, buf1, sem0, sem1):
        wid = lax.axis_index("s") * NC + lax.axis_index("c")
        base = wid * per_w
        pltpu.sync_copy(idx_hbm.at[pl.ds(base, per_w)], idx_v)

        def gather(i, buf, sem):
            return pltpu.async_copy(
                table_hbm.at[idx_v.at[pl.ds(i * CHUNK, CHUNK)]], buf, sem
            )

        def scatter(i, buf):
            pltpu.sync_copy(buf, out_hbm.at[pl.ds(base + i * CHUNK, CHUNK)])

        gather(0, buf0, sem0)

        @pl.loop(0, n_chunks, step=2)
        def _(i):
            gather(i + 1, buf1, sem1)
            pltpu.make_async_copy(table_hbm.at[pl.ds(0, CHUNK)], buf0, sem0).wait()
            scatter(i, buf0)

            @pl.when(i + 2 < n_chunks)
            def _():
                gather(i + 2, buf0, sem0)

            pltpu.make_async_copy(table_hbm.at[pl.ds(0, CHUNK)], buf1, sem1).wait()
            scatter(i + 1, buf1)

    out = gather_k(idx.reshape(n).astype(jnp.int32), table)
    return out.reshape(B, H, D)
